# Initial kernel scaffold; baseline (speedup 1.0000x reference)
import jax
import jax.numpy as jnp
from jax.experimental import pallas as pl


def kernel(x_attr, node_assign, W_pre, b_pre, emb_W, emb_b, W_ops, b_ops, W_res1, b_res1, W_res2, b_res2):
    raise NotImplementedError("write your pallas kernel here")


# speedup vs baseline: 2.0794x; 2.0794x over previous
import functools

import jax
import jax.numpy as jnp
from jax.experimental import pallas as pl


def _elu(x):
    return jnp.where(x > 0, x, jnp.exp(x) - 1.0)


def _attr_kernel(x_ref, a_ref, wpre_ref, bpre_ref, wops_ref, bops_ref,
                 wres1_ref, bres1_ref, wres2_ref, bres2_ref, out_ref, *, n_ops):
    h = jnp.dot(x_ref[...], wpre_ref[...],
                preferred_element_type=jnp.float32) + bpre_ref[...]
    a = a_ref[0]
    acc = jnp.zeros_like(h)
    for k in range(1, n_ops + 1):
        o = jnp.dot(h, wops_ref[k - 1],
                    preferred_element_type=jnp.float32) + bops_ref[k - 1]
        acc = acc + jnp.where(a == k, _elu(o), 0.0)
    r = _elu(jnp.dot(acc, wres1_ref[...],
                     preferred_element_type=jnp.float32) + bres1_ref[...])
    r = _elu(jnp.dot(r, wres2_ref[...],
                     preferred_element_type=jnp.float32) + bres2_ref[...])
    out_ref[...] = _elu(acc + r) + h


def _unattr_kernel(e_ref, a_ref, embb_ref, bops_ref,
                   wres1_ref, bres1_ref, wres2_ref, bres2_ref, out_ref, *, n_ops):
    a = a_ref[0]
    tbl = _elu(bops_ref[...])
    ks = 1 + jax.lax.broadcasted_iota(jnp.int32, (1, n_ops), 1)
    oh = (a == ks).astype(jnp.float32)
    const_part = jnp.dot(oh, tbl, preferred_element_type=jnp.float32)
    emb_part = jnp.where(a == 0, e_ref[...] + embb_ref[...], 0.0)
    h_att = emb_part + const_part
    r = _elu(jnp.dot(h_att, wres1_ref[...],
                     preferred_element_type=jnp.float32) + bres1_ref[...])
    r = _elu(jnp.dot(r, wres2_ref[...],
                     preferred_element_type=jnp.float32) + bres2_ref[...])
    out_ref[...] = _elu(h_att + r)


def kernel(x_attr, node_assign, W_pre, b_pre, emb_W, emb_b, W_ops, b_ops,
           W_res1, b_res1, W_res2, b_res2):
    n_attr, d_in = x_attr.shape
    n_total = node_assign.shape[0]
    n_unattr = n_total - n_attr
    n_ops, d_hid, _ = W_ops.shape
    d_mid = W_res1.shape[1]

    assign = node_assign.astype(jnp.int32)

    B = 512
    pa = pl.cdiv(n_attr, B) * B
    pu = pl.cdiv(n_unattr, B) * B

    x_p = jnp.pad(x_attr, ((0, pa - n_attr), (0, 0)))
    a_attr = jnp.pad(assign[:n_attr], (0, pa - n_attr)).reshape(pa // B, B, 1)
    e_p = jnp.pad(emb_W, ((0, pu - n_unattr), (0, 0)))
    a_un = jnp.pad(assign[n_attr:], (0, pu - n_unattr)).reshape(pu // B, B, 1)

    b_pre2 = b_pre.reshape(1, d_hid)
    emb_b2 = emb_b.reshape(1, d_hid)
    b_res1_2 = b_res1.reshape(1, d_mid)
    b_res2_2 = b_res2.reshape(1, d_hid)

    full = lambda shape: pl.BlockSpec(shape, lambda i: (0,) * len(shape))

    out_attr = pl.pallas_call(
        functools.partial(_attr_kernel, n_ops=n_ops),
        grid=(pa // B,),
        in_specs=[
            pl.BlockSpec((B, d_in), lambda i: (i, 0)),
            pl.BlockSpec((1, B, 1), lambda i: (i, 0, 0)),
            full((d_in, d_hid)),
            full((1, d_hid)),
            full((n_ops, d_hid, d_hid)),
            full((n_ops, d_hid)),
            full((d_hid, d_mid)),
            full((1, d_mid)),
            full((d_mid, d_hid)),
            full((1, d_hid)),
        ],
        out_specs=pl.BlockSpec((B, d_hid), lambda i: (i, 0)),
        out_shape=jax.ShapeDtypeStruct((pa, d_hid), jnp.float32),
    )(x_p, a_attr, W_pre, b_pre2, W_ops, b_ops, W_res1, b_res1_2,
      W_res2, b_res2_2)

    out_unattr = pl.pallas_call(
        functools.partial(_unattr_kernel, n_ops=n_ops),
        grid=(pu // B,),
        in_specs=[
            pl.BlockSpec((B, d_hid), lambda i: (i, 0)),
            pl.BlockSpec((1, B, 1), lambda i: (i, 0, 0)),
            full((1, d_hid)),
            full((n_ops, d_hid)),
            full((d_hid, d_mid)),
            full((1, d_mid)),
            full((d_mid, d_hid)),
            full((1, d_hid)),
        ],
        out_specs=pl.BlockSpec((B, d_hid), lambda i: (i, 0)),
        out_shape=jax.ShapeDtypeStruct((pu, d_hid), jnp.float32),
    )(e_p, a_un, emb_b2, b_ops, W_res1, b_res1_2, W_res2, b_res2_2)

    return jnp.concatenate([out_attr[:n_attr], out_unattr[:n_unattr]], axis=0)
